# baseline (speedup 1.0000x reference)
import functools

import jax
import jax.numpy as jnp
from jax import lax
from jax.experimental import pallas as pl
from jax.experimental.pallas import tpu as pltpu
from jax.experimental.pallas import tpu_sc as plsc

NEGK = -1e30
TS = 40
UP = 80


def _lae(a, b):
    mx = jnp.maximum(a, b)
    d = jnp.abs(a - b)
    return mx + jnp.log1p(jnp.exp(-d))


def _sc_body(x3, meta, so, xbo, xeo, xbuf, mtbuf, os_, oxb, oxe,
             *, B, T, U, H, nslab):
    wid = lax.axis_index("c") * 16 + lax.axis_index("s")
    wpb = 32 // B
    b = wid // wpb
    t0 = (wid % wpb) * nslab
    pltpu.sync_copy(meta.at[b], mtbuf)
    li = lax.broadcasted_iota(jnp.int32, (16,), 0)
    bsp = mtbuf[79]
    zf = jnp.zeros((16,), jnp.float32)

    def cell(u, emit):
        lblsp = mtbuf[u] if emit else None
        acc_s, acc_b, acc_e = zf, zf, zf
        for k in range(H // 16):
            v = xbuf[0, u, pl.ds(16 * k, 16)]
            hk = li + (16 * k)
            acc_s = acc_s + jnp.exp(v)
            acc_b = acc_b + jnp.where(hk == bsp, v, zf)
            if emit:
                acc_e = acc_e + jnp.where(hk == lblsp, v, zf)
        os_[pl.ds(u * 16, 16)] = acc_s
        oxb[pl.ds(u * 16, 16)] = acc_b
        if emit:
            oxe[pl.ds(u * 16, 16)] = acc_e

    def slab(i, _):
        t = t0 + i
        pltpu.sync_copy(x3.at[pl.ds(b * T + t, 1)], xbuf)

        def group(g, _):
            for j in range(16):
                cell(g * 16 + j, True)
            return 0

        lax.fori_loop(0, (U - 1) // 16, group, 0)
        cell(U - 1, False)
        oxe[pl.ds((U - 1) * 16, 16)] = zf
        pltpu.sync_copy(os_, so.at[t, b])
        pltpu.sync_copy(oxb, xbo.at[t, b])
        pltpu.sync_copy(oxe, xeo.at[t, b])
        return 0

    lax.fori_loop(0, nslab, slab, 0)


def _phase1_body(lbl_ref, bi_ref, x_ref, blank_ref, emit_ref, *, U, H, Tb):
    bi = bi_ref[0]
    hi1 = jax.lax.broadcasted_iota(jnp.int32, (1, 1, H), 2)
    lbl = lbl_ref[...]
    hi2 = jax.lax.broadcasted_iota(jnp.int32, (U - 1, H), 1)
    oh = hi2[None] == lbl[:, :, None]
    B = lbl.shape[0]
    for tb in range(Tb):
        xb = x_ref[:, tb, :, :]
        m = jnp.max(xb, axis=-1)
        s = jnp.sum(jnp.exp(xb - m[..., None]), axis=-1)
        lse = m + jnp.log(s)
        blankv = jnp.sum(jnp.where(hi1 == bi, xb, 0.0), axis=-1)
        emitv = jnp.sum(jnp.where(oh, xb[:, :U - 1, :], 0.0), axis=-1)
        blank_ref[tb] = blankv - lse
        emit_ref[tb] = jnp.concatenate(
            [emitv - lse[:, :U - 1], jnp.zeros((B, 1), jnp.float32)], axis=1)


def _skew(src, s0, s1, *, B, U, T, PAD):
    ND = PAD + T + U - 1
    nchunks = (ND - PAD) // PAD
    ui = jax.lax.broadcasted_iota(jnp.int32, (1, B, U), 2)
    s0[pl.ds(0, PAD)] = jnp.zeros((PAD, B, U), jnp.float32)
    s1[pl.ds(0, PAD)] = jnp.zeros((PAD, B, U), jnp.float32)
    s0[pl.ds(PAD, T)] = src
    s0[pl.ds(PAD + T, ND - PAD - T)] = jnp.zeros(
        (ND - PAD - T, B, U), jnp.float32)
    bufs = [s0, s1]
    for step, s in enumerate([1, 2, 4, 8, 16, 32, 64]):
        sr, dst = bufs[step % 2], bufs[(step + 1) % 2]
        mask = (ui & s) != 0
        for c in range(nchunks):
            base = PAD + c * PAD
            cur = sr[pl.ds(base, PAD)]
            sh = sr[pl.ds(base - s, PAD)]
            dst[pl.ds(base, PAD)] = jnp.where(mask, sh, cur)
    return bufs[1]


def _dp_body(s_ref, xb_ref, xe_ref, mm_ref, lpb_ref, lpe_ref,
             yoh_ref, fm2_ref, out_ref, wb0, wb1, we0, we1,
             *, B, T, U, PAD, TSv, UPv):
    ND = T + U - 1
    mm = mm_ref[...]

    def fold(ref):
        v = ref[...].reshape(TSv * B, UPv * 16)
        r = jax.lax.dot_general(v, mm, (((1,), (0,)), ((), ())),
                                preferred_element_type=jnp.float32)
        return r.reshape(TSv, B, UPv)[:, :, :U]

    lse = jnp.log(fold(s_ref))
    lpb_sc = fold(xb_ref) - lse
    lpe_sc = fold(xe_ref) - lse
    lpb = jnp.concatenate([lpb_sc, lpb_ref[...]], axis=0)
    lpe = jnp.concatenate([lpe_sc, lpe_ref[...]], axis=0)
    wb = _skew(lpb, wb0, wb1, B=B, U=U, T=T, PAD=PAD)
    we = _skew(lpe, we0, we1, B=B, U=U, T=T, PAD=PAD)

    yoh = yoh_ref[...]
    ui = jax.lax.broadcasted_iota(jnp.int32, (B, U), 1)
    d0 = jnp.where(ui == 0, 0.0, NEGK)
    sel0 = fm2_ref[0][:, None]
    a_acc = d0 * yoh * sel0
    b_acc = wb[PAD] * yoh * sel0

    def body(d, carry):
        dv, wb_cur, a_acc, b_acc = carry
        wb_next = wb[PAD + d]
        we_prev = we[PAD + d - 1]
        t1 = dv + wb_cur
        t2 = dv + we_prev
        t2s = jnp.concatenate(
            [jnp.full((B, 1), NEGK, jnp.float32), t2[:, :U - 1]], axis=1)
        dn = _lae(t1, t2s)
        sel = fm2_ref[d][:, None]
        a_acc = a_acc + dn * yoh * sel
        b_acc = b_acc + wb_next * yoh * sel
        return dn, wb_next, a_acc, b_acc

    _, _, a_acc, b_acc = lax.fori_loop(
        1, ND, body, (d0, wb[PAD], a_acc, b_acc), unroll=4)
    loss = -(jnp.sum(a_acc + b_acc, axis=1))
    out_ref[...] = loss[None, :]


def kernel(x, label, f_len, y_len, blank_idx):
    B, T, U, H = x.shape
    PAD = 64
    bi = jnp.asarray(blank_idx, jnp.int32).reshape(1)

    x3 = x.reshape(B * T, U, H)
    meta = jnp.zeros((B, 80, 16), jnp.int32)
    meta = meta.at[:, :U - 1, :].set(label[:, :, None])
    meta = meta.at[:, 79, :].set(jnp.asarray(blank_idx, jnp.int32))
    nslab = TS // (32 // B)
    mesh = plsc.VectorSubcoreMesh(core_axis_name="c", subcore_axis_name="s")
    sc = pl.kernel(
        functools.partial(_sc_body, B=B, T=T, U=U, H=H, nslab=nslab),
        out_type=[jax.ShapeDtypeStruct((TS, B, UP * 16), jnp.float32)
                  for _ in range(3)],
        mesh=mesh,
        scratch_types=[
            pltpu.VMEM((1, U, H), jnp.float32),
            pltpu.VMEM((80, 16), jnp.int32),
            pltpu.VMEM((UP * 16,), jnp.float32),
            pltpu.VMEM((UP * 16,), jnp.float32),
            pltpu.VMEM((UP * 16,), jnp.float32),
        ],
    )
    s_sc, xb_sc, xe_sc = sc(x3, meta)

    Tb = 8
    p1 = pl.pallas_call(
        functools.partial(_phase1_body, U=U, H=H, Tb=Tb),
        grid=((T - TS) // Tb,),
        in_specs=[
            pl.BlockSpec((B, U - 1), lambda t: (0, 0)),
            pl.BlockSpec(memory_space=pltpu.SMEM),
            pl.BlockSpec((B, Tb, U, H), lambda t: (0, t + TS // Tb, 0, 0)),
        ],
        out_specs=[
            pl.BlockSpec((Tb, B, U), lambda t: (t, 0, 0)),
            pl.BlockSpec((Tb, B, U), lambda t: (t, 0, 0)),
        ],
        out_shape=[
            jax.ShapeDtypeStruct((T - TS, B, U), jnp.float32),
            jax.ShapeDtypeStruct((T - TS, B, U), jnp.float32),
        ],
    )
    lp_blank_tc, lp_emit_tc = p1(label, bi, x)

    ND = T + U - 1
    yoh = (jax.lax.broadcasted_iota(jnp.int32, (B, U), 1)
           == y_len[:, None]).astype(jnp.float32)
    fm2 = (jax.lax.broadcasted_iota(jnp.int32, (ND, B), 0)
           == (f_len - 1 + y_len)[None, :]).astype(jnp.float32)

    mm = (jax.lax.broadcasted_iota(jnp.int32, (UP * 16, UP), 0) // 16
          == jax.lax.broadcasted_iota(jnp.int32, (UP * 16, UP), 1)
          ).astype(jnp.float32)
    NB = PAD + ND
    dp = pl.pallas_call(
        functools.partial(_dp_body, B=B, T=T, U=U, PAD=PAD, TSv=TS, UPv=UP),
        out_shape=jax.ShapeDtypeStruct((1, B), jnp.float32),
        scratch_shapes=[pltpu.VMEM((NB, B, U), jnp.float32)
                        for _ in range(4)],
    )
    loss = dp(s_sc, xb_sc, xe_sc, mm, lp_blank_tc, lp_emit_tc, yoh, fm2)
    return loss.reshape(B)


# speedup vs baseline: 1.3069x; 1.3069x over previous
import functools

import jax
import jax.numpy as jnp
from jax import lax
from jax.experimental import pallas as pl
from jax.experimental.pallas import tpu as pltpu
from jax.experimental.pallas import tpu_sc as plsc

NEGK = -1e30
TS = 8
UP = 80


def _lae(a, b):
    mx = jnp.maximum(a, b)
    d = jnp.abs(a - b)
    return mx + jnp.log1p(jnp.exp(-d))


def _sc_body(x3, meta, so, xbo, xeo, xbuf, mtbuf, os_, oxb, oxe,
             *, B, T, U, H, nslab):
    wid = lax.axis_index("c") * 16 + lax.axis_index("s")
    wpb = 32 // B
    b = wid // wpb
    t0 = (wid % wpb) * nslab
    pltpu.sync_copy(meta.at[b], mtbuf)
    li = lax.broadcasted_iota(jnp.int32, (16,), 0)
    bsp = mtbuf[79]
    zf = jnp.zeros((16,), jnp.float32)

    def cell(u, emit):
        lblsp = mtbuf[u] if emit else None
        acc_s, acc_b, acc_e = zf, zf, zf
        for k in range(H // 16):
            v = xbuf[0, u, pl.ds(16 * k, 16)]
            hk = li + (16 * k)
            acc_s = acc_s + jnp.exp(v)
            acc_b = acc_b + jnp.where(hk == bsp, v, zf)
            if emit:
                acc_e = acc_e + jnp.where(hk == lblsp, v, zf)
        os_[pl.ds(u * 16, 16)] = acc_s
        oxb[pl.ds(u * 16, 16)] = acc_b
        if emit:
            oxe[pl.ds(u * 16, 16)] = acc_e

    def slab(i, _):
        t = t0 + i
        pltpu.sync_copy(x3.at[pl.ds(b * T + t, 1)], xbuf)

        def group(g, _):
            for j in range(16):
                cell(g * 16 + j, True)
            return 0

        lax.fori_loop(0, (U - 1) // 16, group, 0)
        cell(U - 1, False)
        oxe[pl.ds((U - 1) * 16, 16)] = zf
        pltpu.sync_copy(os_, so.at[t, b])
        pltpu.sync_copy(oxb, xbo.at[t, b])
        pltpu.sync_copy(oxe, xeo.at[t, b])
        return 0

    lax.fori_loop(0, nslab, slab, 0)


def _phase1_body(lbl_ref, bi_ref, x_ref, blank_ref, emit_ref, *, U, H, Tb):
    bi = bi_ref[0]
    hi1 = jax.lax.broadcasted_iota(jnp.int32, (1, 1, H), 2)
    lbl = lbl_ref[...]
    hi2 = jax.lax.broadcasted_iota(jnp.int32, (U - 1, H), 1)
    oh = hi2[None] == lbl[:, :, None]
    B = lbl.shape[0]
    for tb in range(Tb):
        xb = x_ref[:, tb, :, :]
        m = jnp.max(xb, axis=-1)
        s = jnp.sum(jnp.exp(xb - m[..., None]), axis=-1)
        lse = m + jnp.log(s)
        blankv = jnp.sum(jnp.where(hi1 == bi, xb, 0.0), axis=-1)
        emitv = jnp.sum(jnp.where(oh, xb[:, :U - 1, :], 0.0), axis=-1)
        blank_ref[tb] = blankv - lse
        emit_ref[tb] = jnp.concatenate(
            [emitv - lse[:, :U - 1], jnp.zeros((B, 1), jnp.float32)], axis=1)


def _skew(src, s0, s1, *, B, U, T, PAD):
    ND = PAD + T + U - 1
    nchunks = (ND - PAD) // PAD
    ui = jax.lax.broadcasted_iota(jnp.int32, (1, B, U), 2)
    s0[pl.ds(0, PAD)] = jnp.zeros((PAD, B, U), jnp.float32)
    s1[pl.ds(0, PAD)] = jnp.zeros((PAD, B, U), jnp.float32)
    s0[pl.ds(PAD, T)] = src
    s0[pl.ds(PAD + T, ND - PAD - T)] = jnp.zeros(
        (ND - PAD - T, B, U), jnp.float32)
    bufs = [s0, s1]
    for step, s in enumerate([1, 2, 4, 8, 16, 32, 64]):
        sr, dst = bufs[step % 2], bufs[(step + 1) % 2]
        mask = (ui & s) != 0
        for c in range(nchunks):
            base = PAD + c * PAD
            cur = sr[pl.ds(base, PAD)]
            sh = sr[pl.ds(base - s, PAD)]
            dst[pl.ds(base, PAD)] = jnp.where(mask, sh, cur)
    return bufs[1]


def _dp_body(s_ref, xb_ref, xe_ref, mm_ref, lpb_ref, lpe_ref,
             yoh_ref, fm2_ref, out_ref, wb0, wb1, we0, we1,
             *, B, T, U, PAD, TSv, UPv):
    ND = T + U - 1
    mm = mm_ref[...]

    def fold(ref):
        v = ref[...].reshape(TSv * B, UPv * 16)
        r = jax.lax.dot_general(v, mm, (((1,), (0,)), ((), ())),
                                preferred_element_type=jnp.float32)
        return r.reshape(TSv, B, UPv)[:, :, :U]

    lse = jnp.log(fold(s_ref))
    lpb_sc = fold(xb_ref) - lse
    lpe_sc = fold(xe_ref) - lse
    lpb = jnp.concatenate([lpb_sc, lpb_ref[...]], axis=0)
    lpe = jnp.concatenate([lpe_sc, lpe_ref[...]], axis=0)
    wb = _skew(lpb, wb0, wb1, B=B, U=U, T=T, PAD=PAD)
    we = _skew(lpe, we0, we1, B=B, U=U, T=T, PAD=PAD)

    yoh = yoh_ref[...]
    ui = jax.lax.broadcasted_iota(jnp.int32, (B, U), 1)
    d0 = jnp.where(ui == 0, 0.0, NEGK)
    sel0 = fm2_ref[0][:, None]
    a_acc = d0 * yoh * sel0
    b_acc = wb[PAD] * yoh * sel0

    def body(d, carry):
        dv, wb_cur, a_acc, b_acc = carry
        wb_next = wb[PAD + d]
        we_prev = we[PAD + d - 1]
        t1 = dv + wb_cur
        t2 = dv + we_prev
        t2s = jnp.concatenate(
            [jnp.full((B, 1), NEGK, jnp.float32), t2[:, :U - 1]], axis=1)
        dn = _lae(t1, t2s)
        sel = fm2_ref[d][:, None]
        a_acc = a_acc + dn * yoh * sel
        b_acc = b_acc + wb_next * yoh * sel
        return dn, wb_next, a_acc, b_acc

    _, _, a_acc, b_acc = lax.fori_loop(
        1, ND, body, (d0, wb[PAD], a_acc, b_acc), unroll=4)
    loss = -(jnp.sum(a_acc + b_acc, axis=1))
    out_ref[...] = loss[None, :]


def kernel(x, label, f_len, y_len, blank_idx):
    B, T, U, H = x.shape
    PAD = 64
    bi = jnp.asarray(blank_idx, jnp.int32).reshape(1)

    x3 = x.reshape(B * T, U, H)
    meta = jnp.zeros((B, 80, 16), jnp.int32)
    meta = meta.at[:, :U - 1, :].set(label[:, :, None])
    meta = meta.at[:, 79, :].set(jnp.asarray(blank_idx, jnp.int32))
    nslab = TS // (32 // B)
    mesh = plsc.VectorSubcoreMesh(core_axis_name="c", subcore_axis_name="s")
    sc = pl.kernel(
        functools.partial(_sc_body, B=B, T=T, U=U, H=H, nslab=nslab),
        out_type=[jax.ShapeDtypeStruct((TS, B, UP * 16), jnp.float32)
                  for _ in range(3)],
        mesh=mesh,
        scratch_types=[
            pltpu.VMEM((1, U, H), jnp.float32),
            pltpu.VMEM((80, 16), jnp.int32),
            pltpu.VMEM((UP * 16,), jnp.float32),
            pltpu.VMEM((UP * 16,), jnp.float32),
            pltpu.VMEM((UP * 16,), jnp.float32),
        ],
    )
    s_sc, xb_sc, xe_sc = sc(x3, meta)

    Tb = 8
    p1 = pl.pallas_call(
        functools.partial(_phase1_body, U=U, H=H, Tb=Tb),
        grid=((T - TS) // Tb,),
        in_specs=[
            pl.BlockSpec((B, U - 1), lambda t: (0, 0)),
            pl.BlockSpec(memory_space=pltpu.SMEM),
            pl.BlockSpec((B, Tb, U, H), lambda t: (0, t + TS // Tb, 0, 0)),
        ],
        out_specs=[
            pl.BlockSpec((Tb, B, U), lambda t: (t, 0, 0)),
            pl.BlockSpec((Tb, B, U), lambda t: (t, 0, 0)),
        ],
        out_shape=[
            jax.ShapeDtypeStruct((T - TS, B, U), jnp.float32),
            jax.ShapeDtypeStruct((T - TS, B, U), jnp.float32),
        ],
    )
    lp_blank_tc, lp_emit_tc = p1(label, bi, x)

    ND = T + U - 1
    yoh = (jax.lax.broadcasted_iota(jnp.int32, (B, U), 1)
           == y_len[:, None]).astype(jnp.float32)
    fm2 = (jax.lax.broadcasted_iota(jnp.int32, (ND, B), 0)
           == (f_len - 1 + y_len)[None, :]).astype(jnp.float32)

    mm = (jax.lax.broadcasted_iota(jnp.int32, (UP * 16, UP), 0) // 16
          == jax.lax.broadcasted_iota(jnp.int32, (UP * 16, UP), 1)
          ).astype(jnp.float32)
    NB = PAD + ND
    dp = pl.pallas_call(
        functools.partial(_dp_body, B=B, T=T, U=U, PAD=PAD, TSv=TS, UPv=UP),
        out_shape=jax.ShapeDtypeStruct((1, B), jnp.float32),
        scratch_shapes=[pltpu.VMEM((NB, B, U), jnp.float32)
                        for _ in range(4)],
    )
    loss = dp(s_sc, xb_sc, xe_sc, mm, lp_blank_tc, lp_emit_tc, yoh, fm2)
    return loss.reshape(B)


# speedup vs baseline: 1.3161x; 1.0071x over previous
import functools

import jax
import jax.numpy as jnp
from jax import lax
from jax.experimental import pallas as pl
from jax.experimental.pallas import tpu as pltpu
from jax.experimental.pallas import tpu_sc as plsc

NEGK = -1e30
TS = 4
UP = 80


def _lae(a, b):
    mx = jnp.maximum(a, b)
    d = jnp.abs(a - b)
    return mx + jnp.log1p(jnp.exp(-d))


def _sc_body(x3, meta, so, xbo, xeo, xbuf, mtbuf, os_, oxb, oxe,
             *, B, T, U, H, nslab):
    wid = lax.axis_index("c") * 16 + lax.axis_index("s")
    wpb = 32 // B
    b = wid // wpb
    t0 = (wid % wpb) * nslab
    pltpu.sync_copy(meta.at[b], mtbuf)
    li = lax.broadcasted_iota(jnp.int32, (16,), 0)
    bsp = mtbuf[79]
    zf = jnp.zeros((16,), jnp.float32)

    def cell(u, emit):
        lblsp = mtbuf[u] if emit else None
        acc_s, acc_b, acc_e = zf, zf, zf
        for k in range(H // 16):
            v = xbuf[0, u, pl.ds(16 * k, 16)]
            hk = li + (16 * k)
            acc_s = acc_s + jnp.exp(v)
            acc_b = acc_b + jnp.where(hk == bsp, v, zf)
            if emit:
                acc_e = acc_e + jnp.where(hk == lblsp, v, zf)
        os_[pl.ds(u * 16, 16)] = acc_s
        oxb[pl.ds(u * 16, 16)] = acc_b
        if emit:
            oxe[pl.ds(u * 16, 16)] = acc_e

    def slab(i, _):
        t = t0 + i
        pltpu.sync_copy(x3.at[pl.ds(b * T + t, 1)], xbuf)

        def group(g, _):
            for j in range(16):
                cell(g * 16 + j, True)
            return 0

        lax.fori_loop(0, (U - 1) // 16, group, 0)
        cell(U - 1, False)
        oxe[pl.ds((U - 1) * 16, 16)] = zf
        pltpu.sync_copy(os_, so.at[t, b])
        pltpu.sync_copy(oxb, xbo.at[t, b])
        pltpu.sync_copy(oxe, xeo.at[t, b])
        return 0

    lax.fori_loop(0, nslab, slab, 0)


def _phase1_body(lbl_ref, bi_ref, x_ref, blank_ref, emit_ref, *, U, H, Tb):
    bi = bi_ref[0]
    hi1 = jax.lax.broadcasted_iota(jnp.int32, (1, 1, H), 2)
    lbl = lbl_ref[...]
    hi2 = jax.lax.broadcasted_iota(jnp.int32, (U - 1, H), 1)
    oh = hi2[None] == lbl[:, :, None]
    B = lbl.shape[0]
    for tb in range(Tb):
        xb = x_ref[:, tb, :, :]
        m = jnp.max(xb, axis=-1)
        s = jnp.sum(jnp.exp(xb - m[..., None]), axis=-1)
        lse = m + jnp.log(s)
        blankv = jnp.sum(jnp.where(hi1 == bi, xb, 0.0), axis=-1)
        emitv = jnp.sum(jnp.where(oh, xb[:, :U - 1, :], 0.0), axis=-1)
        blank_ref[tb] = blankv - lse
        emit_ref[tb] = jnp.concatenate(
            [emitv - lse[:, :U - 1], jnp.zeros((B, 1), jnp.float32)], axis=1)


def _skew(src, s0, s1, *, B, U, T, PAD):
    ND = PAD + T + U - 1
    nchunks = (ND - PAD) // PAD
    ui = jax.lax.broadcasted_iota(jnp.int32, (1, B, U), 2)
    s0[pl.ds(0, PAD)] = jnp.zeros((PAD, B, U), jnp.float32)
    s1[pl.ds(0, PAD)] = jnp.zeros((PAD, B, U), jnp.float32)
    s0[pl.ds(PAD, T)] = src
    s0[pl.ds(PAD + T, ND - PAD - T)] = jnp.zeros(
        (ND - PAD - T, B, U), jnp.float32)
    bufs = [s0, s1]
    for step, s in enumerate([1, 2, 4, 8, 16, 32, 64]):
        sr, dst = bufs[step % 2], bufs[(step + 1) % 2]
        mask = (ui & s) != 0
        for c in range(nchunks):
            base = PAD + c * PAD
            cur = sr[pl.ds(base, PAD)]
            sh = sr[pl.ds(base - s, PAD)]
            dst[pl.ds(base, PAD)] = jnp.where(mask, sh, cur)
    return bufs[1]


def _dp_body(s_ref, xb_ref, xe_ref, mm_ref, lpb_ref, lpe_ref,
             yoh_ref, fm2_ref, out_ref, wb0, wb1, we0, we1,
             *, B, T, U, PAD, TSv, UPv):
    ND = T + U - 1
    mm = mm_ref[...]

    def fold(ref):
        v = ref[...].reshape(TSv * B, UPv * 16)
        r = jax.lax.dot_general(v, mm, (((1,), (0,)), ((), ())),
                                preferred_element_type=jnp.float32)
        return r.reshape(TSv, B, UPv)[:, :, :U]

    lse = jnp.log(fold(s_ref))
    lpb_sc = fold(xb_ref) - lse
    lpe_sc = fold(xe_ref) - lse
    lpb = jnp.concatenate([lpb_sc, lpb_ref[...]], axis=0)
    lpe = jnp.concatenate([lpe_sc, lpe_ref[...]], axis=0)
    wb = _skew(lpb, wb0, wb1, B=B, U=U, T=T, PAD=PAD)
    we = _skew(lpe, we0, we1, B=B, U=U, T=T, PAD=PAD)

    yoh = yoh_ref[...]
    ui = jax.lax.broadcasted_iota(jnp.int32, (B, U), 1)
    d0 = jnp.where(ui == 0, 0.0, NEGK)
    sel0 = fm2_ref[0][:, None]
    a_acc = d0 * yoh * sel0
    b_acc = wb[PAD] * yoh * sel0

    def body(d, carry):
        dv, wb_cur, a_acc, b_acc = carry
        wb_next = wb[PAD + d]
        we_prev = we[PAD + d - 1]
        t1 = dv + wb_cur
        t2 = dv + we_prev
        t2s = jnp.concatenate(
            [jnp.full((B, 1), NEGK, jnp.float32), t2[:, :U - 1]], axis=1)
        dn = _lae(t1, t2s)
        sel = fm2_ref[d][:, None]
        a_acc = a_acc + dn * yoh * sel
        b_acc = b_acc + wb_next * yoh * sel
        return dn, wb_next, a_acc, b_acc

    _, _, a_acc, b_acc = lax.fori_loop(
        1, ND, body, (d0, wb[PAD], a_acc, b_acc), unroll=4)
    loss = -(jnp.sum(a_acc + b_acc, axis=1))
    out_ref[...] = loss[None, :]


def kernel(x, label, f_len, y_len, blank_idx):
    B, T, U, H = x.shape
    PAD = 64
    bi = jnp.asarray(blank_idx, jnp.int32).reshape(1)

    x3 = x.reshape(B * T, U, H)
    meta = jnp.zeros((B, 80, 16), jnp.int32)
    meta = meta.at[:, :U - 1, :].set(label[:, :, None])
    meta = meta.at[:, 79, :].set(jnp.asarray(blank_idx, jnp.int32))
    nslab = TS // (32 // B)
    mesh = plsc.VectorSubcoreMesh(core_axis_name="c", subcore_axis_name="s")
    sc = pl.kernel(
        functools.partial(_sc_body, B=B, T=T, U=U, H=H, nslab=nslab),
        out_type=[jax.ShapeDtypeStruct((TS, B, UP * 16), jnp.float32)
                  for _ in range(3)],
        mesh=mesh,
        scratch_types=[
            pltpu.VMEM((1, U, H), jnp.float32),
            pltpu.VMEM((80, 16), jnp.int32),
            pltpu.VMEM((UP * 16,), jnp.float32),
            pltpu.VMEM((UP * 16,), jnp.float32),
            pltpu.VMEM((UP * 16,), jnp.float32),
        ],
    )
    s_sc, xb_sc, xe_sc = sc(x3, meta)

    Tb = 8
    p1 = pl.pallas_call(
        functools.partial(_phase1_body, U=U, H=H, Tb=Tb),
        grid=((T - TS) // Tb,),
        in_specs=[
            pl.BlockSpec((B, U - 1), lambda t: (0, 0)),
            pl.BlockSpec(memory_space=pltpu.SMEM),
            pl.BlockSpec((B, Tb, U, H), lambda t: (0, t + TS // Tb, 0, 0)),
        ],
        out_specs=[
            pl.BlockSpec((Tb, B, U), lambda t: (t, 0, 0)),
            pl.BlockSpec((Tb, B, U), lambda t: (t, 0, 0)),
        ],
        out_shape=[
            jax.ShapeDtypeStruct((T - TS, B, U), jnp.float32),
            jax.ShapeDtypeStruct((T - TS, B, U), jnp.float32),
        ],
    )
    lp_blank_tc, lp_emit_tc = p1(label, bi, x)

    ND = T + U - 1
    yoh = (jax.lax.broadcasted_iota(jnp.int32, (B, U), 1)
           == y_len[:, None]).astype(jnp.float32)
    fm2 = (jax.lax.broadcasted_iota(jnp.int32, (ND, B), 0)
           == (f_len - 1 + y_len)[None, :]).astype(jnp.float32)

    mm = (jax.lax.broadcasted_iota(jnp.int32, (UP * 16, UP), 0) // 16
          == jax.lax.broadcasted_iota(jnp.int32, (UP * 16, UP), 1)
          ).astype(jnp.float32)
    NB = PAD + ND
    dp = pl.pallas_call(
        functools.partial(_dp_body, B=B, T=T, U=U, PAD=PAD, TSv=TS, UPv=UP),
        out_shape=jax.ShapeDtypeStruct((1, B), jnp.float32),
        scratch_shapes=[pltpu.VMEM((NB, B, U), jnp.float32)
                        for _ in range(4)],
    )
    loss = dp(s_sc, xb_sc, xe_sc, mm, lp_blank_tc, lp_emit_tc, yoh, fm2)
    return loss.reshape(B)
